# Initial kernel scaffold; baseline (speedup 1.0000x reference)
import jax
import jax.numpy as jnp
from jax.experimental import pallas as pl


def kernel(x, ring, params):
    raise NotImplementedError("write your pallas kernel here")


# speedup vs baseline: 7.1061x; 7.1061x over previous
import functools

import jax
import jax.numpy as jnp
from jax import lax
from jax.experimental import pallas as pl

EPS = 1e-5
F32 = jnp.float32
BF16 = jnp.bfloat16
NEG = -jnp.inf

_MM = (((1,), (0,)), ((), ()))


def _dot1(a, b, dn=_MM):
    return lax.dot_general(a.astype(BF16), b.astype(BF16), dn,
                           preferred_element_type=F32)


def _dotx(a, b, dn=_MM):
    ah = a.astype(BF16)
    al = (a - ah.astype(F32)).astype(BF16)
    bh = b.astype(BF16)
    f = lambda x, y: lax.dot_general(x, y, dn, preferred_element_type=F32)
    return f(ah, bh) + f(al, bh)


def _norm_lane(z, g, b):
    m = jnp.mean(z, axis=1, keepdims=True)
    v = jnp.mean((z - m) * (z - m), axis=1, keepdims=True)
    return g * (z - m) / jnp.sqrt(v + EPS) + b


def _norm_row(z, g, b):
    m = jnp.mean(z, axis=0, keepdims=True)
    v = jnp.mean((z - m) * (z - m), axis=0, keepdims=True)
    return g * (z - m) / jnp.sqrt(v + EPS) + b


def _stn_body(xin, c1W, bn1g, bn1b, c2W, bn2g, bn2b, c3W, bn3g, bn3b,
              f1W, bn4g, bn4b, f2W, bn5g, bn5b, f3W, f3bi, out):
    X = xin[:]
    P = X.shape[1]
    NB = P // 2048
    z1 = _dot1(c1W[:], X)
    h1 = jax.nn.relu(_norm_lane(z1, bn1g[:], bn1b[:]))
    z2 = _dot1(c2W[:], h1)
    h2 = jax.nn.relu(_norm_lane(z2, bn2g[:], bn2b[:]))
    s3 = jnp.zeros((1024, 1), F32)
    ss3 = jnp.zeros((1024, 1), F32)
    mxl, mnl = [], []
    for b in range(NB):
        z3 = _dot1(c3W[:], h2[:, 2048 * b:2048 * (b + 1)])
        s3 = s3 + jnp.sum(z3, axis=1, keepdims=True)
        ss3 = ss3 + jnp.sum(z3 * z3, axis=1, keepdims=True)
        mxl.append(jnp.max(z3, axis=1, keepdims=True))
        mnl.append(jnp.min(z3, axis=1, keepdims=True))
    m3 = s3 / P
    v3 = jnp.maximum(ss3 / P - m3 * m3, 0.0)
    sc3 = bn3g[:] / jnp.sqrt(v3 + EPS)
    mx = jnp.concatenate(mxl, axis=1)
    mn = jnp.concatenate(mnl, axis=1)
    hm = jax.nn.relu(jnp.where(sc3 >= 0, sc3 * (mx - m3), sc3 * (mn - m3))
                     + bn3b[:])
    h4 = _dot1(hm, f1W[:], (((0,), (1,)), ((), ())))
    h4 = jax.nn.relu(_norm_row(h4, bn4g[:], bn4b[:]))
    h5 = _dot1(h4, f2W[:], (((1,), (1,)), ((), ())))
    h5 = jax.nn.relu(_norm_row(h5, bn5g[:], bn5b[:]))
    h6 = _dot1(h5, f3W[:], (((1,), (1,)), ((), ())))
    out[:] = h6 + f3bi[:]


def _trunk_body(xin, trans, c1W, bn1g, bn1b, gW1, gbn1g, gbn1b,
                gW2, gbn2g, gbn2b, Wall, rgT, rbT,
                pf_out, gt_out, pool_out):
    X = xin[:]
    P = X.shape[1]
    NB = P // 2048
    NR = 16
    bh = (lax.broadcasted_iota(jnp.int32, (NB, P), 1) // 2048
          == lax.broadcasted_iota(jnp.int32, (NB, P), 0)).astype(F32)
    T9 = _dotx(trans[:], bh, (((0,), (0,)), ((), ())))
    T9b = T9.astype(BF16).astype(F32)
    Xb = X[0:3, :].astype(BF16).astype(F32)
    rows = []
    for i in range(3):
        acc = T9b[3 * i:3 * i + 1, :] * Xb[0:1, :]
        for j in range(1, 3):
            acc = acc + T9b[3 * i + j:3 * i + j + 1, :] * Xb[j:j + 1, :]
        rows.append(acc)
    xc = jnp.concatenate(rows + [X[3:5, :], jnp.zeros((3, P), F32)], axis=0)
    zp = _dot1(c1W[:], xc)
    PF = jax.nn.relu(_norm_lane(zp, bn1g[:], bn1b[:]))
    pf_out[:] = PF
    zg1 = _dot1(gW1[:], PF)
    G1 = jax.nn.relu(_norm_lane(zg1, gbn1g[:], gbn1b[:]))

    sg = jnp.zeros((1024, 1), F32)
    ssg = jnp.zeros((1024, 1), F32)
    gmx, gmn = [], []
    rs = jnp.zeros((128, NR), F32)
    rss = jnp.zeros((128, NR), F32)
    cnt = jnp.zeros((1, NR), F32)
    ones_row = jnp.ones((1, 2048), F32)
    riota = lax.broadcasted_iota(jnp.int32, (NR, 2048), 0).astype(F32)
    bmxl, bmnl = [], []
    for b in range(NB):
        sl = slice(2048 * b, 2048 * (b + 1))
        zg2 = _dot1(gW2[:], G1[:, sl])
        sg = sg + jnp.sum(zg2, axis=1, keepdims=True)
        ssg = ssg + jnp.sum(zg2 * zg2, axis=1, keepdims=True)
        gmx.append(jnp.max(zg2, axis=1, keepdims=True))
        gmn.append(jnp.min(zg2, axis=1, keepdims=True))
        YA = _dot1(Wall[:], PF[:, sl])
        OH = (jnp.broadcast_to(X[5:6, sl], (NR, 2048)) == riota).astype(F32)
        Y = YA[0:128, :] * OH[0:1, :]
        for r in range(1, NR):
            Y = Y + YA[128 * r:128 * (r + 1), :] * OH[r:r + 1, :]
        rs = rs + _dotx(Y, OH, (((1,), (1,)), ((), ())))
        rss = rss + _dotx(Y * Y, OH, (((1,), (1,)), ((), ())))
        cnt = cnt + _dotx(ones_row, OH, (((1,), (1,)), ((), ())))
        mcols, ncols = [], []
        for r in range(NR):
            msk = OH[r:r + 1, :] > 0.5
            mcols.append(jnp.max(jnp.where(msk, Y, NEG), axis=1,
                                 keepdims=True))
            ncols.append(jnp.min(jnp.where(msk, Y, -NEG), axis=1,
                                 keepdims=True))
        bmxl.append(jnp.concatenate(mcols, axis=1))
        bmnl.append(jnp.concatenate(ncols, axis=1))

    mg = sg / P
    vg = jnp.maximum(ssg / P - mg * mg, 0.0)
    scg = gbn2g[:] / jnp.sqrt(vg + EPS)
    gmxC = jnp.concatenate(gmx, axis=1)
    gmnC = jnp.concatenate(gmn, axis=1)
    gt_out[:] = (jnp.where(scg >= 0, scg * (gmxC - mg), scg * (gmnC - mg))
                 + gbn2b[:])

    cntc = jnp.maximum(cnt, 1.0)
    rmean = rs / cntc
    rvar = jnp.maximum(rss / cntc - rmean * rmean, 0.0)
    rsc = rgT[:] / jnp.sqrt(rvar + EPS)
    pcols = []
    for b in range(NB):
        pb = (jnp.where(rsc >= 0, rsc * (bmxl[b] - rmean),
                        rsc * (bmnl[b] - rmean)) + rbT[:])
        pb = jnp.where(bmxl[b] == NEG, 0.0, pb)
        pcols.append(pb)
    pool_out[:] = jnp.concatenate(pcols, axis=1)


def _asm_body(xin, pf, pool, gt, out):
    b = pl.program_id(0)
    NR = 16
    out[0, 0:64, :] = pf[:]
    ringv = xin[5:6, :]
    riota = lax.broadcasted_iota(jnp.int32, (NR, 2048), 0).astype(F32)
    OH = (jnp.broadcast_to(ringv, (NR, 2048)) == riota).astype(F32)
    j0 = lax.broadcasted_iota(jnp.int32, (128, NR), 0)
    j1 = lax.broadcasted_iota(jnp.int32, (128, NR), 1)
    RM = (j0 - j1 == NR * b).astype(F32)
    pooled = _dotx(pool[:], RM)
    rf = pooled[:, 0:1] * OH[0:1, :]
    for r in range(1, NR):
        rf = rf + pooled[:, r:r + 1] * OH[r:r + 1, :]
    out[0, 64:192, :] = rf
    bm = (lax.broadcasted_iota(jnp.int32, (8, 2048), 0) == b).astype(F32)
    out[0, 192:1216, :] = _dotx(gt[:], bm)


def kernel(x, ring, params):
    B, C, N = x.shape
    P = B * N
    NR = params['rW'].shape[0]
    xcm = x.transpose(1, 0, 2).reshape(C, P)
    ringrow = ring.reshape(1, P).astype(F32)
    xin = jnp.concatenate([xcm, ringrow, jnp.zeros((2, P), F32)], axis=0)

    p = params['stn']
    col = lambda a: a[:, None]
    row = lambda a: a[None, :]
    c1Wp = jnp.pad(p['c1W'], ((0, 0), (0, 3)))
    f3Wp = jnp.pad(p['f3W'], ((0, 7), (0, 0)))
    iden = jnp.eye(3, dtype=F32).reshape(9)
    f3bi = row(jnp.pad(p['f3b'] + iden, (0, 7)))

    trans = pl.pallas_call(
        _stn_body,
        out_shape=jax.ShapeDtypeStruct((B, 16), F32),
    )(xin, c1Wp, col(p['bn1g']), col(p['bn1b']), p['c2W'],
      col(p['bn2g']), col(p['bn2b']), p['c3W'], col(p['bn3g']),
      col(p['bn3b']), p['f1W'], row(p['bn4g']), row(p['bn4b']),
      p['f2W'], row(p['bn5g']), row(p['bn5b']), f3Wp, f3bi)

    c1Wm = jnp.pad(params['c1W'], ((0, 0), (0, 3)))
    Wall = params['rW'].reshape(NR * 128, 64)
    PF, Gt, POOL = pl.pallas_call(
        _trunk_body,
        out_shape=[jax.ShapeDtypeStruct((64, P), F32),
                   jax.ShapeDtypeStruct((1024, B), F32),
                   jax.ShapeDtypeStruct((128, B * NR), F32)],
    )(xin, trans, c1Wm, col(params['bn1g']), col(params['bn1b']),
      params['gW1'], col(params['gbn1g']), col(params['gbn1b']),
      params['gW2'], col(params['gbn2g']), col(params['gbn2b']),
      Wall, params['rg'].T, params['rbeta'].T)

    out = pl.pallas_call(
        _asm_body,
        grid=(B,),
        in_specs=[
            pl.BlockSpec((8, 2048), lambda b: (0, b)),
            pl.BlockSpec((64, 2048), lambda b: (0, b)),
            pl.BlockSpec((128, B * NR), lambda b: (0, 0)),
            pl.BlockSpec((1024, B), lambda b: (0, 0)),
        ],
        out_specs=pl.BlockSpec((1, 1216, 2048), lambda b: (b, 0, 0)),
        out_shape=jax.ShapeDtypeStruct((B, 1216, 2048), F32),
    )(xin, PF, POOL, Gt)
    return out
